# baseline (speedup 1.0000x reference)
import functools

import jax
import jax.numpy as jnp
from jax import lax
from jax.experimental import pallas as pl
from jax.experimental.pallas import tpu as pltpu
from jax.experimental.pallas import tpu_sc as plsc

D = 128
NC = 2
NS = 16
NW = NC * NS
CHUNK = 128
NBUF = 6
LAG = 2


@functools.partial(jax.jit, static_argnums=(2, 3))
def _emb_lookup(idx, table, n_chunks, per_w):
    mesh = plsc.VectorSubcoreMesh(core_axis_name="c", subcore_axis_name="s")
    total = NW * per_w

    @functools.partial(
        pl.kernel,
        out_type=jax.ShapeDtypeStruct((total, D), jnp.float32),
        mesh=mesh,
        scratch_types=[
            pltpu.VMEM((n_chunks, CHUNK), jnp.int32),
            [pltpu.VMEM((CHUNK, D), jnp.float32) for _ in range(NBUF)],
            [pltpu.SemaphoreType.DMA for _ in range(NBUF)],
            [pltpu.SemaphoreType.DMA for _ in range(NBUF)],
        ],
    )
    def emb(idx_hbm, table_hbm, out_hbm, idx_v, rows, gsem, osem):
        wid = lax.axis_index("s") * NC + lax.axis_index("c")
        row_base = wid * per_w
        pltpu.sync_copy(idx_hbm.at[wid], idx_v)

        def fire_gather(j, b):
            pltpu.async_copy(table_hbm.at[idx_v.at[j]], rows[b], gsem[b])

        def wait_gather(j, b):
            pltpu.make_async_copy(
                table_hbm.at[idx_v.at[j]], rows[b], gsem[b]
            ).wait()

        def fire_out(j, b):
            pltpu.async_copy(
                rows[b],
                out_hbm.at[pl.ds(row_base + j * CHUNK, CHUNK)],
                osem[b],
            )

        def wait_out(j, b):
            pltpu.make_async_copy(
                rows[b],
                out_hbm.at[pl.ds(row_base + j * CHUNK, CHUNK)],
                osem[b],
            ).wait()

        for b in range(NBUF):
            fire_gather(b, b)
        for j in range(NBUF):
            wait_gather(j, j)
            fire_out(j, j)
            if j >= LAG:
                wait_out(j - LAG, j - LAG)
                fire_gather(j - LAG + NBUF, j - LAG)

        n_groups = (n_chunks - NBUF) // NBUF - 1

        @pl.loop(0, n_groups)
        def _group(g):
            j0 = NBUF + g * NBUF
            for b in range(NBUF):
                j = j0 + b
                wait_gather(j, b)
                fire_out(j, b)
                jd = j - LAG
                bd = (b - LAG) % NBUF
                wait_out(jd, bd)
                fire_gather(jd + NBUF, bd)

        for jj in range(NBUF * (n_groups + 1), n_chunks):
            b = jj % NBUF
            wait_gather(jj, b)
            fire_out(jj, b)
            jd = jj - LAG
            wait_out(jd, jd % NBUF)
            if jd + NBUF < n_chunks:
                fire_gather(jd + NBUF, jd % NBUF)
        for jd in range(n_chunks - LAG, n_chunks):
            wait_out(jd, jd % NBUF)

    return emb(idx, table)


def kernel(label_ids, table):
    B, S = label_ids.shape
    total = B * S
    per_w = total // NW
    n_chunks = per_w // CHUNK
    idx = label_ids.reshape(NW, n_chunks, CHUNK).astype(jnp.int32)
    out = _emb_lookup(idx, table, n_chunks, per_w)
    return out.reshape(B, S, D)
